# baseline (speedup 1.0000x reference)
import functools

import jax
import jax.numpy as jnp
from jax import lax
from jax.experimental import pallas as pl

N_ATOMS = 320000
NUM_BLOCKS = 32000
NUM_GRAPHS = 64
HIDDEN = 128
N_LEVELS = 50

ATILE = 1280
NTILES = N_ATOMS // ATILE


def _silu(x):
    return x * jax.nn.sigmoid(x)


def _fused_body(side_ref, h_ref, wenc_ref, wpos_ref, wout_ref,
                pred_ref, gacc_ref):
    i = pl.program_id(0)

    @pl.when(i == 0)
    def _():
        gacc_ref[...] = jnp.zeros_like(gacc_ref)

    side = side_ref[...]
    zp = side[:, 0:3]
    wb = side[:, 3:4]
    gid = side[:, 4:5]
    x = jnp.dot(h_ref[...], wenc_ref[...], preferred_element_type=jnp.float32)
    x = x + jnp.dot(zp, wpos_ref[...], preferred_element_type=jnp.float32)
    u = _silu(x)
    pred_ref[...] = jnp.dot(u, wout_ref[...], preferred_element_type=jnp.float32)
    onehot = (gid.astype(jnp.int32)
              == lax.broadcasted_iota(jnp.int32, (ATILE, NUM_GRAPHS), 1))
    wu = wb * u
    gacc_ref[...] += lax.dot_general(
        onehot.astype(jnp.float32), wu,
        dimension_numbers=(((0,), (0,)), ((), ())),
        preferred_element_type=jnp.float32)


def _finalize_body(spred_ref, cnt_ref, noise_ref, gacc_ref, w1_ref, b1_ref,
                   w2_ref, b2_ref, energy_ref, loss_ref):
    c = cnt_ref[...]
    m = (c > 0.0).astype(jnp.float32)
    d = spred_ref[...] / jnp.maximum(c, 1.0) + noise_ref[...] * m
    loss_ref[...] = (jnp.sum(d * d) / (NUM_BLOCKS * 3.0)).reshape(1, 1)
    hg = _silu(jnp.dot(gacc_ref[...], w1_ref[...],
                       preferred_element_type=jnp.float32) + b1_ref[...])
    energy_ref[...] = jnp.dot(hg, w2_ref[...],
                              preferred_element_type=jnp.float32) + b2_ref[...]


@jax.jit
def kernel(Z, H, noise, sigmas, W_enc, W_pos, W_out, W1, b1, W2, b2,
           block_id, batch_id, noise_level):
    f32 = jnp.float32
    start = jnp.searchsorted(block_id, jnp.arange(NUM_BLOCKS + 1, dtype=jnp.int32))
    c = (start[1:] - start[:-1]).astype(f32)
    gstart = jnp.searchsorted(batch_id, jnp.arange(NUM_GRAPHS + 1, dtype=jnp.int32))
    nb = (gstart[1:] - gstart[:-1]).astype(f32)
    sig = sigmas[noise_level][batch_id]
    t = noise * sig[:, None]
    wb = 1.0 / (jnp.maximum(c, 1.0) * jnp.maximum(nb, 1.0)[batch_id])
    table = jnp.concatenate(
        [t, wb[:, None], batch_id.astype(f32)[:, None],
         jnp.zeros((NUM_BLOCKS, 3), f32)], axis=1)
    g8 = table[block_id]
    side = jnp.concatenate([Z + g8[:, 0:3], g8[:, 3:5]], axis=1)

    pred, graph_repr = pl.pallas_call(
        _fused_body,
        grid=(NTILES,),
        in_specs=[
            pl.BlockSpec((ATILE, 5), lambda i: (i, 0)),
            pl.BlockSpec((ATILE, HIDDEN), lambda i: (i, 0)),
            pl.BlockSpec((HIDDEN, HIDDEN), lambda i: (0, 0)),
            pl.BlockSpec((3, HIDDEN), lambda i: (0, 0)),
            pl.BlockSpec((HIDDEN, 3), lambda i: (0, 0)),
        ],
        out_specs=[
            pl.BlockSpec((ATILE, 3), lambda i: (i, 0)),
            pl.BlockSpec((NUM_GRAPHS, HIDDEN), lambda i: (0, 0)),
        ],
        out_shape=[
            jax.ShapeDtypeStruct((N_ATOMS, 3), f32),
            jax.ShapeDtypeStruct((NUM_GRAPHS, HIDDEN), f32),
        ],
    )(side, H, W_enc, W_pos, W_out)

    s_pred = jax.ops.segment_sum(pred, block_id, num_segments=NUM_BLOCKS)

    energy2, loss2 = pl.pallas_call(
        _finalize_body,
        out_shape=[
            jax.ShapeDtypeStruct((NUM_GRAPHS, 1), f32),
            jax.ShapeDtypeStruct((1, 1), f32),
        ],
    )(s_pred.T, c[None, :], noise.T, graph_repr, W1, b1[None, :], W2,
      b2[None, :])

    return energy2[:, 0], graph_repr, loss2[0, 0]
